# baseline (speedup 1.0000x reference)
import functools

import jax
import jax.numpy as jnp
from jax import lax
from jax.experimental import pallas as pl
from jax.experimental.pallas import tpu as pltpu
from jax.experimental.pallas import tpu_sc as plsc

N = 10000
D = 128
H = D // 2
E = 320000
COEFF = 0.1
NC = 2
NS = 16
C = 128
NCHUNK = 158
E2 = NS * NCHUNK * C
NP = 10240
RPT = NP // NS


def _prepass_body(z_ref, nf_ref, a_ref, b_ref):
    zr = z_ref[...] * nf_ref[...]
    a_ref[...] = zr[:, :H]
    b_ref[...] = zr[:, H:]


def _prepass(z, nf):
    blk = 1000
    return pl.pallas_call(
        _prepass_body,
        grid=(N // blk,),
        in_specs=[
            pl.BlockSpec((blk, D), lambda i: (i, 0)),
            pl.BlockSpec((blk, 1), lambda i: (i, 0)),
        ],
        out_specs=[
            pl.BlockSpec((blk, H), lambda i: (i, 0)),
            pl.BlockSpec((blk, H), lambda i: (i, 0)),
        ],
        out_shape=[
            jax.ShapeDtypeStruct((N, H), jnp.float32),
            jax.ShapeDtypeStruct((N, H), jnp.float32),
        ],
    )(z, nf)


def _postpass_body(z_ref, nf_ref, sa_ref, sb_ref, out_ref):
    s = jnp.concatenate([sa_ref[...], sb_ref[...]], axis=1)
    out_ref[...] = z_ref[...] - (2.0 * COEFF / N) * nf_ref[...] * s


def _postpass(z, nf, sa, sb):
    blk = 1000
    return pl.pallas_call(
        _postpass_body,
        grid=(N // blk,),
        in_specs=[
            pl.BlockSpec((blk, D), lambda i: (i, 0)),
            pl.BlockSpec((blk, 1), lambda i: (i, 0)),
            pl.BlockSpec((blk, H), lambda i: (i, 0)),
            pl.BlockSpec((blk, H), lambda i: (i, 0)),
        ],
        out_specs=pl.BlockSpec((blk, D), lambda i: (i, 0)),
        out_shape=jax.ShapeDtypeStruct((N, D), jnp.float32),
    )(z, nf, sa, sb)


def _diff(buf_a, buf_b):
    def body(i, _):
        for f16 in range(H // 16):
            sl = pl.ds(f16 * 16, 16)
            a = buf_a[i, sl]
            b = buf_b[i, sl]
            buf_a[i, sl] = a - b
            buf_b[i, sl] = b - a
        return 0

    lax.fori_loop(0, C, body, 0, unroll=2)


def _sc_body(zra, zrb, rows3, cols3, zeros,
             sa, sb,
             idx_r, idx_c, a0, b0, a1, b1, acc,
             isem, g0, g1, s0, s1):
    c = lax.axis_index("c")
    s = lax.axis_index("s")

    pi = pltpu.async_copy(rows3.at[s], idx_r, isem)
    pc = pltpu.async_copy(cols3.at[s], idx_c, isem)
    r0 = s * RPT
    pltpu.sync_copy(zeros.at[pl.ds(r0, RPT)], acc.at[pl.ds(r0, RPT)])
    pi.wait()
    pc.wait()
    plsc.subcore_barrier()

    def run_edges(table):
        @pl.loop(0, NCHUNK, step=2)
        def _(k):
            ir0, ic0 = idx_r.at[k], idx_c.at[k]
            ir1, ic1 = idx_r.at[k + 1], idx_c.at[k + 1]
            ga0 = pltpu.async_copy(table.at[ir0], a0, g0)
            gb0 = pltpu.async_copy(table.at[ic0], b0, g0)
            ga1 = pltpu.async_copy(table.at[ir1], a1, g1)
            gb1 = pltpu.async_copy(table.at[ic1], b1, g1)
            ga0.wait()
            gb0.wait()
            _diff(a0, b0)
            sa0 = pltpu.async_copy(a0, acc.at[ir0], s0, add=True)
            sb0 = pltpu.async_copy(b0, acc.at[ic0], s0, add=True)
            ga1.wait()
            gb1.wait()
            _diff(a1, b1)
            sa1 = pltpu.async_copy(a1, acc.at[ir1], s1, add=True)
            sb1 = pltpu.async_copy(b1, acc.at[ic1], s1, add=True)
            sa0.wait()
            sb0.wait()
            sa1.wait()
            sb1.wait()

    @pl.when(c == 0)
    def _():
        run_edges(zra)

    @pl.when(c == 1)
    def _():
        run_edges(zrb)

    plsc.subcore_barrier()

    @pl.when(c == 0)
    def _():
        pltpu.sync_copy(acc.at[pl.ds(r0, RPT)], sa.at[pl.ds(r0, RPT)])

    @pl.when(c == 1)
    def _():
        pltpu.sync_copy(acc.at[pl.ds(r0, RPT)], sb.at[pl.ds(r0, RPT)])


_sc_kernel = functools.partial(
    pl.kernel,
    out_type=[
        jax.ShapeDtypeStruct((NP, H), jnp.float32),
        jax.ShapeDtypeStruct((NP, H), jnp.float32),
    ],
    mesh=plsc.VectorSubcoreMesh(
        core_axis_name="c", subcore_axis_name="s",
        num_cores=NC, num_subcores=NS,
    ),
    compiler_params=pltpu.CompilerParams(use_tc_tiling_on_sc=False),
    scratch_types=[
        pltpu.VMEM((NCHUNK, C), jnp.int32),
        pltpu.VMEM((NCHUNK, C), jnp.int32),
        pltpu.VMEM((C, H), jnp.float32),
        pltpu.VMEM((C, H), jnp.float32),
        pltpu.VMEM((C, H), jnp.float32),
        pltpu.VMEM((C, H), jnp.float32),
        pltpu.VMEM_SHARED((NP, H), jnp.float32),
        pltpu.SemaphoreType.DMA,
        pltpu.SemaphoreType.DMA,
        pltpu.SemaphoreType.DMA,
        pltpu.SemaphoreType.DMA,
        pltpu.SemaphoreType.DMA,
    ],
)(_sc_body)


@jax.jit
def kernel(z, x, edge_index, norm_factor):
    del x
    zra, zrb = _prepass(z, norm_factor)
    pad = jnp.zeros((E2 - E,), jnp.int32)
    rows3 = jnp.concatenate([edge_index[0], pad]).reshape(NS, NCHUNK, C)
    cols3 = jnp.concatenate([edge_index[1], pad]).reshape(NS, NCHUNK, C)
    zeros = jnp.zeros((NP, H), jnp.float32)
    sa, sb = _sc_kernel(zra, zrb, rows3, cols3, zeros)
    return _postpass(z, norm_factor, sa, sb)


# speedup vs baseline: 1.1247x; 1.1247x over previous
import functools

import jax
import jax.numpy as jnp
from jax import lax
from jax.experimental import pallas as pl
from jax.experimental.pallas import tpu as pltpu
from jax.experimental.pallas import tpu_sc as plsc

N = 10000
D = 128
E = 320000
COEFF = 0.1
NC = 2
NS = 16
C = 88
NCHUNK = 114
E2 = NC * NS * NCHUNK * C
NP = 10240
RPT = NP // NS


def _prepass_body(z_ref, nf_ref, zr_ref):
    zr_ref[...] = z_ref[...] * nf_ref[...]


def _prepass(z, nf):
    blk = 1000
    return pl.pallas_call(
        _prepass_body,
        grid=(N // blk,),
        in_specs=[
            pl.BlockSpec((blk, D), lambda i: (i, 0)),
            pl.BlockSpec((blk, 1), lambda i: (i, 0)),
        ],
        out_specs=pl.BlockSpec((blk, D), lambda i: (i, 0)),
        out_shape=jax.ShapeDtypeStruct((N, D), jnp.float32),
    )(z, nf)


def _postpass_body(z_ref, nf_ref, sa_ref, sb_ref, out_ref):
    s = sa_ref[...] + sb_ref[...]
    out_ref[...] = z_ref[...] - (2.0 * COEFF / N) * nf_ref[...] * s


def _postpass(z, nf, sa, sb):
    blk = 1000
    return pl.pallas_call(
        _postpass_body,
        grid=(N // blk,),
        in_specs=[
            pl.BlockSpec((blk, D), lambda i: (i, 0)),
            pl.BlockSpec((blk, 1), lambda i: (i, 0)),
            pl.BlockSpec((blk, D), lambda i: (i, 0)),
            pl.BlockSpec((blk, D), lambda i: (i, 0)),
        ],
        out_specs=pl.BlockSpec((blk, D), lambda i: (i, 0)),
        out_shape=jax.ShapeDtypeStruct((N, D), jnp.float32),
    )(z, nf, sa, sb)


def _diff(buf_a, buf_b):
    def body(i, _):
        for f16 in range(D // 16):
            sl = pl.ds(f16 * 16, 16)
            a = buf_a[i, sl]
            b = buf_b[i, sl]
            buf_a[i, sl] = a - b
            buf_b[i, sl] = b - a
        return 0

    lax.fori_loop(0, C, body, 0, unroll=2)


def _sc_body(zr, rows, cols, zeros,
             sa, sb,
             ir0, ic0, ir1, ic1, a0, b0, a1, b1, acc,
             isem, g0, g1, s0, s1):
    c = lax.axis_index("c")
    s = lax.axis_index("s")
    wid = c * NS + s
    base = wid * NCHUNK * C

    r0 = s * RPT
    pltpu.sync_copy(zeros.at[pl.ds(r0, RPT)], acc.at[pl.ds(r0, RPT)])
    plsc.subcore_barrier()

    @pl.loop(0, NCHUNK, step=2)
    def _(k):
        off0 = base + k * C
        off1 = off0 + C
        i0 = pltpu.async_copy(rows.at[pl.ds(off0, C)], ir0, isem)
        i1 = pltpu.async_copy(cols.at[pl.ds(off0, C)], ic0, isem)
        i2 = pltpu.async_copy(rows.at[pl.ds(off1, C)], ir1, isem)
        i3 = pltpu.async_copy(cols.at[pl.ds(off1, C)], ic1, isem)
        i0.wait()
        i1.wait()
        ga0 = pltpu.async_copy(zr.at[ir0], a0, g0)
        gb0 = pltpu.async_copy(zr.at[ic0], b0, g0)
        i2.wait()
        i3.wait()
        ga1 = pltpu.async_copy(zr.at[ir1], a1, g1)
        gb1 = pltpu.async_copy(zr.at[ic1], b1, g1)
        ga0.wait()
        gb0.wait()
        _diff(a0, b0)
        sa0 = pltpu.async_copy(a0, acc.at[ir0], s0, add=True)
        sb0 = pltpu.async_copy(b0, acc.at[ic0], s0, add=True)
        ga1.wait()
        gb1.wait()
        _diff(a1, b1)
        sa1 = pltpu.async_copy(a1, acc.at[ir1], s1, add=True)
        sb1 = pltpu.async_copy(b1, acc.at[ic1], s1, add=True)
        sa0.wait()
        sb0.wait()
        sa1.wait()
        sb1.wait()

    plsc.subcore_barrier()

    @pl.when(c == 0)
    def _():
        pltpu.sync_copy(acc.at[pl.ds(r0, RPT)], sa.at[pl.ds(r0, RPT)])

    @pl.when(c == 1)
    def _():
        pltpu.sync_copy(acc.at[pl.ds(r0, RPT)], sb.at[pl.ds(r0, RPT)])


_sc_kernel = functools.partial(
    pl.kernel,
    out_type=[
        jax.ShapeDtypeStruct((NP, D), jnp.float32),
        jax.ShapeDtypeStruct((NP, D), jnp.float32),
    ],
    mesh=plsc.VectorSubcoreMesh(
        core_axis_name="c", subcore_axis_name="s",
        num_cores=NC, num_subcores=NS,
    ),
    scratch_types=[
        pltpu.VMEM((C,), jnp.int32),
        pltpu.VMEM((C,), jnp.int32),
        pltpu.VMEM((C,), jnp.int32),
        pltpu.VMEM((C,), jnp.int32),
        pltpu.VMEM((C, D), jnp.float32),
        pltpu.VMEM((C, D), jnp.float32),
        pltpu.VMEM((C, D), jnp.float32),
        pltpu.VMEM((C, D), jnp.float32),
        pltpu.VMEM_SHARED((NP, D), jnp.float32),
        pltpu.SemaphoreType.DMA,
        pltpu.SemaphoreType.DMA,
        pltpu.SemaphoreType.DMA,
        pltpu.SemaphoreType.DMA,
        pltpu.SemaphoreType.DMA,
    ],
)(_sc_body)


@jax.jit
def kernel(z, x, edge_index, norm_factor):
    del x
    zr = _prepass(z, norm_factor)
    pad = jnp.zeros((E2 - E,), jnp.int32)
    rows1 = jnp.concatenate([edge_index[0], pad])
    cols1 = jnp.concatenate([edge_index[1], pad])
    zeros = jnp.zeros((NP, D), jnp.float32)
    sa, sb = _sc_kernel(zr, rows1, cols1, zeros)
    return _postpass(z, norm_factor, sa, sb)
